# baseline (speedup 1.0000x reference)
import functools

import jax
import jax.numpy as jnp
from jax import lax
from jax.experimental import pallas as pl
from jax.experimental.pallas import tpu as pltpu
from jax.experimental.pallas import tpu_sc as plsc

N = 10000
E = 320000
H = 128
R = 10
L = 6
G = 64
OUT = 32

NB = 1000
C = 128
NCHUNK = E // C
NRP = 100352
ZW = NRP // 16
NPT = 624

_mesh = plsc.VectorSubcoreMesh(core_axis_name="c", subcore_axis_name="s")


def _count_body(dst_hbm, et_hbm, cnt_out, cnt_sh, dstv, etv, idxv, ones, zbuf, sem):
    cid = lax.axis_index("c")
    sid = lax.axis_index("s")
    wid = cid * 16 + sid

    def _zb(i, _):
        zbuf[pl.ds(i * 16, 16)] = jnp.zeros((16,), jnp.float32)
        return 0

    lax.fori_loop(0, ZW // 16, _zb, 0)
    for k in range(C // 16):
        ones[pl.ds(k * 16, 16)] = jnp.ones((16,), jnp.float32)
    pltpu.sync_copy(zbuf, cnt_sh.at[pl.ds(sid * ZW, ZW)])
    plsc.subcore_barrier()

    ebase = (78 * wid + jnp.minimum(wid, 4)) * C
    extra = (wid < 4).astype(jnp.int32)
    nch = 78 + extra
    pltpu.sync_copy(dst_hbm.at[pl.ds(ebase, 78 * C)], dstv.at[pl.ds(0, 78 * C)])
    pltpu.sync_copy(et_hbm.at[pl.ds(ebase, 78 * C)], etv.at[pl.ds(0, 78 * C)])

    @pl.when(wid < 4)
    def _():
        pltpu.sync_copy(dst_hbm.at[pl.ds(ebase + 78 * C, C)],
                        dstv.at[pl.ds(78 * C, C)])
        pltpu.sync_copy(et_hbm.at[pl.ds(ebase + 78 * C, C)],
                        etv.at[pl.ds(78 * C, C)])

    def _idx(t, _):
        sl = pl.ds(t * 16, 16)
        idxv[sl] = dstv[sl] * R + etv[sl]
        return 0

    lax.fori_loop(0, nch * (C // 16), _idx, 0)

    def _fire(c):
        pltpu.async_copy(ones, cnt_sh.at[idxv.at[pl.ds(c * C, C)]], sem,
                         add=True)

    def _drain(c):
        pltpu.make_async_copy(ones, cnt_sh.at[idxv.at[pl.ds(c * C, C)]],
                              sem).wait()

    def _sc(c, _):
        _fire(c)

        @pl.when(c >= 13)
        def _():
            _drain(c - 13)

        return 0

    lax.fori_loop(0, nch, _sc, 0)

    def _dr(c, _):
        _drain(nch - 13 + c)
        return 0

    lax.fori_loop(0, 13, _dr, 0)
    plsc.subcore_barrier()
    pltpu.sync_copy(cnt_sh.at[pl.ds(sid * ZW, ZW)],
                    cnt_out.at[pl.ds(cid * NRP + sid * ZW, ZW)])


_count_edges = functools.partial(
    pl.kernel,
    out_type=jax.ShapeDtypeStruct((2 * NRP,), jnp.float32),
    mesh=_mesh,
    scratch_types=[
        pltpu.VMEM_SHARED((NRP,), jnp.float32),
        pltpu.VMEM((79 * C,), jnp.int32),
        pltpu.VMEM((79 * C,), jnp.int32),
        pltpu.VMEM((79 * C,), jnp.int32),
        pltpu.VMEM((C,), jnp.float32),
        pltpu.VMEM((ZW,), jnp.float32),
        pltpu.SemaphoreType.DMA,
    ],
)(_count_body)


def _weights_body(src_hbm, dst_hbm, et_hbm, inv_hbm, w_out, gidx_out,
                  inv_v, srcb, dstb, etb, wb, gb):
    cid = lax.axis_index("c")
    sid = lax.axis_index("s")
    wid = cid * 16 + sid
    pltpu.sync_copy(inv_hbm, inv_v)

    ebase = (78 * wid + jnp.minimum(wid, 4)) * C

    def _blk(blk, nw):
        pltpu.sync_copy(src_hbm.at[pl.ds(blk, nw)], srcb.at[pl.ds(0, nw)])
        pltpu.sync_copy(dst_hbm.at[pl.ds(blk, nw)], dstb.at[pl.ds(0, nw)])
        pltpu.sync_copy(et_hbm.at[pl.ds(blk, nw)], etb.at[pl.ds(0, nw)])

        def _t(t, _):
            sl = pl.ds(t * 16, 16)
            sv = srcb[sl]
            dv = dstb[sl]
            ev = etb[sl]
            wb[sl] = plsc.load_gather(inv_v, [dv * R + ev])
            gb[sl] = ev * N + sv
            return 0

        lax.fori_loop(0, nw // 16, _t, 0)
        pltpu.sync_copy(wb.at[pl.ds(0, nw)], w_out.at[pl.ds(blk, nw)])
        pltpu.sync_copy(gb.at[pl.ds(0, nw)], gidx_out.at[pl.ds(blk, nw)])

    for b in range(3):
        _blk(ebase + b * BW, BW)

    @pl.when(wid < 4)
    def _():
        _blk(ebase + 3 * BW, C)


_edge_weights = functools.partial(
    pl.kernel,
    out_type=(jax.ShapeDtypeStruct((E,), jnp.float32),
              jax.ShapeDtypeStruct((E,), jnp.int32)),
    mesh=_mesh,
    scratch_types=[
        pltpu.VMEM((NRP,), jnp.float32),
        pltpu.VMEM((26 * C,), jnp.int32),
        pltpu.VMEM((26 * C,), jnp.int32),
        pltpu.VMEM((26 * C,), jnp.int32),
        pltpu.VMEM((26 * C,), jnp.float32),
        pltpu.VMEM((26 * C,), jnp.int32),
    ],
    compiler_params=pltpu.CompilerParams(needs_layout_passes=False),
)(_weights_body)


NPB = 26
BW = NPB * C


def _scatter_body(hr, gidx_hbm, dst_hbm, w_hbm, accs,
                  acc_sh, idxb, dstb, wb, rows0, rows1,
                  sg0, sg1, ss0, ss1):
    cid = lax.axis_index("c")
    sid = lax.axis_index("s")
    wid = cid * 16 + sid

    def _zb(i, _):
        for k in range(8):
            rows0[i, pl.ds(k * 16, 16)] = jnp.zeros((16,), jnp.float32)
        return 0

    lax.fori_loop(0, C, _zb, 0)
    r0 = sid * NPT
    for m in range(4):
        pltpu.sync_copy(rows0, acc_sh.at[pl.ds(r0 + m * C, C)])
    pltpu.sync_copy(rows0.at[pl.ds(0, NPT - 4 * C)],
                    acc_sh.at[pl.ds(r0 + 4 * C, NPT - 4 * C)])

    @pl.when(sid == 15)
    def _():
        pltpu.sync_copy(rows0.at[pl.ds(0, 16)],
                        acc_sh.at[pl.ds(16 * NPT, 16)])

    plsc.subcore_barrier()

    def _scale(rows, woff):
        @plsc.parallel_loop(0, C // 16, 1, unroll=2)
        def _g(g):
            wrow = wb[pl.ds(woff + g * 16, 16)]
            for lane in range(16):
                wv = wrow[lane]
                i = g * 16 + lane
                for k in range(8):
                    sl = pl.ds(k * 16, 16)
                    rows[i, sl] = rows[i, sl] * wv

    def _gather(c, rows, sem):
        return pltpu.make_async_copy(hr.at[idxb.at[pl.ds(c * C, C)]],
                                     rows, sem)

    def _scatter(c, rows, sem):
        return pltpu.async_copy(rows, acc_sh.at[dstb.at[pl.ds(c * C, C)]],
                                sem, add=True)

    def _scatter_wait(c, rows, sem):
        pltpu.make_async_copy(rows, acc_sh.at[dstb.at[pl.ds(c * C, C)]],
                              sem).wait()

    ebase = (78 * wid + jnp.minimum(wid, 4)) * C

    for b in range(3):
        blk = ebase + b * BW
        pltpu.sync_copy(gidx_hbm.at[pl.ds(blk, BW)], idxb)
        pltpu.sync_copy(dst_hbm.at[pl.ds(blk, BW)], dstb)
        pltpu.sync_copy(w_hbm.at[pl.ds(blk, BW)], wb)
        _gather(0, rows0, sg0).start()

        def _pair(p, _):
            c0 = 2 * p
            c1 = c0 + 1

            @pl.when(p > 0)
            def _():
                _scatter_wait(c0 - 1, rows1, ss1)

            _gather(c1, rows1, sg1).start()
            _gather(c0, rows0, sg0).wait()
            _scale(rows0, c0 * C)
            _scatter(c0, rows0, ss0)
            _gather(c1, rows1, sg1).wait()
            _scale(rows1, c1 * C)
            _scatter(c1, rows1, ss1)
            _scatter_wait(c0, rows0, ss0)

            @pl.when(p < NPB // 2 - 1)
            def _():
                _gather(c0 + 2, rows0, sg0).start()

            return 0

        lax.fori_loop(0, NPB // 2, _pair, 0)
        _scatter_wait(NPB - 1, rows1, ss1)

    @pl.when(wid < 4)
    def _():
        blk = ebase + 3 * BW
        pltpu.sync_copy(gidx_hbm.at[pl.ds(blk, C)], idxb.at[pl.ds(0, C)])
        pltpu.sync_copy(dst_hbm.at[pl.ds(blk, C)], dstb.at[pl.ds(0, C)])
        pltpu.sync_copy(w_hbm.at[pl.ds(blk, C)], wb.at[pl.ds(0, C)])
        _gather(0, rows0, sg0).start()
        _gather(0, rows0, sg0).wait()
        _scale(rows0, 0)
        _scatter(0, rows0, ss0)
        _scatter_wait(0, rows0, ss0)

    plsc.subcore_barrier()
    pltpu.sync_copy(acc_sh.at[pl.ds(r0, NPT)], accs.at[cid, pl.ds(r0, NPT)])

    @pl.when(sid == 15)
    def _():
        pltpu.sync_copy(acc_sh.at[pl.ds(16 * NPT, 16)],
                        accs.at[cid, pl.ds(16 * NPT, 16)])


_edge_scatter = functools.partial(
    pl.kernel,
    out_type=jax.ShapeDtypeStruct((2, N, H), jnp.float32),
    mesh=_mesh,
    scratch_types=[
        pltpu.VMEM_SHARED((N, H), jnp.float32),
        pltpu.VMEM((BW,), jnp.int32),
        pltpu.VMEM((BW,), jnp.int32),
        pltpu.VMEM((BW,), jnp.float32),
        pltpu.VMEM((C, H), jnp.float32),
        pltpu.VMEM((C, H), jnp.float32),
        pltpu.SemaphoreType.DMA,
        pltpu.SemaphoreType.DMA,
        pltpu.SemaphoreType.DMA,
        pltpu.SemaphoreType.DMA,
    ],
    compiler_params=pltpu.CompilerParams(needs_layout_passes=False),
)(_scatter_body)


def _inv_body(cnt_ref, inv_ref):
    c = cnt_ref[0] + cnt_ref[1]
    inv_ref[...] = 1.0 / jnp.maximum(c, 1.0)


def _inv_counts(cnt2d):
    return pl.pallas_call(
        _inv_body,
        out_shape=jax.ShapeDtypeStruct((NRP,), jnp.float32),
    )(cnt2d)


def _ln_relu_res(acc0, acc1, root, bcv, gam, bet, hprev):
    s = acc0 + acc1 + root + bcv
    mu = jnp.mean(s, axis=-1, keepdims=True)
    var = jnp.mean((s - mu) ** 2, axis=-1, keepdims=True)
    s = (s - mu) * lax.rsqrt(var + 1e-5) * gam + bet
    return jnp.maximum(s, 0.0) + hprev


def _transforms(h, wrel_ref, wroot_ref, hr_ref, root_ref):
    for r in range(R):
        hr_ref[r] = jnp.dot(h, wrel_ref[r], preferred_element_type=jnp.float32)
    root_ref[...] = jnp.dot(h, wroot_ref[...], preferred_element_type=jnp.float32)


def _dense_in_body(x_ref, wp_ref, bp_ref, wrel_ref, wroot_ref,
                   h_ref, hr_ref, root_ref):
    h = jnp.dot(x_ref[...], wp_ref[...],
                preferred_element_type=jnp.float32) + bp_ref[...]
    h_ref[...] = h
    _transforms(h, wrel_ref, wroot_ref, hr_ref, root_ref)


def _dense_mid_body(acc0_ref, acc1_ref, rootin_ref, hprev_ref, bcv_ref,
                    gam_ref, bet_ref, wrel_ref, wroot_ref,
                    h_ref, hr_ref, root_ref):
    h = _ln_relu_res(acc0_ref[...], acc1_ref[...], rootin_ref[...],
                     bcv_ref[...], gam_ref[...], bet_ref[...], hprev_ref[...])
    h_ref[...] = h
    _transforms(h, wrel_ref, wroot_ref, hr_ref, root_ref)


def _dense_out_body(acc0_ref, acc1_ref, rootin_ref, hprev_ref, bcv_ref,
                    gam_ref, bet_ref, batch_ref,
                    wm1_ref, bm1_ref, wm2_ref, bm2_ref,
                    wv1_ref, bv1_ref, wv2_ref, bv2_ref,
                    logits_ref, value_ref, g_acc):
    i = pl.program_id(0)

    @pl.when(i == 0)
    def _():
        g_acc[...] = jnp.zeros_like(g_acc)

    h = _ln_relu_res(acc0_ref[...], acc1_ref[...], rootin_ref[...],
                     bcv_ref[...], gam_ref[...], bet_ref[...], hprev_ref[...])
    b = batch_ref[...].reshape(NB)
    oh = (lax.broadcasted_iota(jnp.int32, (G, NB), 0) == b[None, :])
    g_acc[...] += jnp.dot(oh.astype(jnp.float32), h,
                          preferred_element_type=jnp.float32)

    @pl.when(i == pl.num_programs(0) - 1)
    def _():
        g = g_acc[...]
        hm = jnp.maximum(jnp.dot(g, wm1_ref[...],
                                 preferred_element_type=jnp.float32)
                         + bm1_ref[...], 0.0)
        logits_ref[...] = jnp.dot(hm, wm2_ref[...],
                                  preferred_element_type=jnp.float32) + bm2_ref[...]
        hv = jnp.maximum(jnp.dot(g, wv1_ref[...],
                                 preferred_element_type=jnp.float32)
                         + bv1_ref[...], 0.0)
        value_ref[...] = jnp.dot(hv, wv2_ref[...],
                                 preferred_element_type=jnp.float32) + bv2_ref[...]


_row_spec = pl.BlockSpec((NB, H), lambda i: (i, 0))
_vecH_spec = pl.BlockSpec((H,), lambda i: (0,))
_wrel_spec = pl.BlockSpec((R, H, H), lambda i: (0, 0, 0))
_wHH_spec = pl.BlockSpec((H, H), lambda i: (0, 0))
_hr_spec = pl.BlockSpec((R, NB, H), lambda i: (0, i, 0))

_h_hr_root_shapes = (jax.ShapeDtypeStruct((N, H), jnp.float32),
                     jax.ShapeDtypeStruct((R, N, H), jnp.float32),
                     jax.ShapeDtypeStruct((N, H), jnp.float32))


def _dense_in(x, Wp, bp, Wrel0, Wroot0):
    return pl.pallas_call(
        _dense_in_body,
        grid=(N // NB,),
        in_specs=[_row_spec, _wHH_spec, _vecH_spec, _wrel_spec, _wHH_spec],
        out_specs=[_row_spec, _hr_spec, _row_spec],
        out_shape=_h_hr_root_shapes,
    )(x, Wp, bp, Wrel0, Wroot0)


def _dense_mid(acc0, acc1, root, hprev, bcv, gam, bet, Wrel_l, Wroot_l):
    return pl.pallas_call(
        _dense_mid_body,
        grid=(N // NB,),
        in_specs=[_row_spec, _row_spec, _row_spec, _row_spec,
                  _vecH_spec, _vecH_spec, _vecH_spec, _wrel_spec, _wHH_spec],
        out_specs=[_row_spec, _hr_spec, _row_spec],
        out_shape=_h_hr_root_shapes,
    )(acc0, acc1, root, hprev, bcv, gam, bet, Wrel_l, Wroot_l)


def _dense_out(acc0, acc1, root, hprev, bcv, gam, bet, batch3d,
               Wm1, bm1, Wm2, bm2, Wv1, bv1, Wv2, bv2):
    wH = pl.BlockSpec((H, H), lambda i: (0, 0))
    wO = pl.BlockSpec((H, OUT), lambda i: (0, 0))
    vO = pl.BlockSpec((OUT,), lambda i: (0,))
    out_spec = pl.BlockSpec((G, OUT), lambda i: (0, 0))
    return pl.pallas_call(
        _dense_out_body,
        grid=(N // NB,),
        in_specs=[_row_spec, _row_spec, _row_spec, _row_spec,
                  _vecH_spec, _vecH_spec, _vecH_spec,
                  pl.BlockSpec((1, 1, NB), lambda i: (i, 0, 0)),
                  wH, _vecH_spec, wO, vO, wH, _vecH_spec, wO, vO],
        out_specs=[out_spec, out_spec],
        out_shape=(jax.ShapeDtypeStruct((G, OUT), jnp.float32),
                   jax.ShapeDtypeStruct((G, OUT), jnp.float32)),
        scratch_shapes=[pltpu.VMEM((G, H), jnp.float32)],
    )(acc0, acc1, root, hprev, bcv, gam, bet, batch3d,
      Wm1, bm1, Wm2, bm2, Wv1, bv1, Wv2, bv2)


def kernel(x, edge_index, edge_type, batch, Wp, bp, Wrel, Wroot, bconv,
           gamma, beta, Wm1, bm1, Wm2, bm2, Wv1, bv1, Wv2, bv2):
    src = edge_index[0].astype(jnp.int32)
    dst = edge_index[1].astype(jnp.int32)
    et = edge_type.astype(jnp.int32)
    batch3d = batch.astype(jnp.int32).reshape(N // NB, 1, NB)

    cnt = _count_edges(dst, et)
    inv = _inv_counts(cnt.reshape(2, NRP))
    w, gidx = _edge_weights(src, dst, et, inv)

    h, hr, root = _dense_in(x, Wp, bp, Wrel[0], Wroot[0])
    for l in range(L):
        accs = _edge_scatter(hr.reshape(R * N, H), gidx, dst, w)
        if l < L - 1:
            h, hr, root = _dense_mid(accs[0], accs[1], root, h, bconv[l],
                                     gamma[l], beta[l], Wrel[l + 1],
                                     Wroot[l + 1])
        else:
            logits, value = _dense_out(accs[0], accs[1], root, h, bconv[l],
                                       gamma[l], beta[l], batch3d,
                                       Wm1, bm1, Wm2, bm2, Wv1, bv1, Wv2, bv2)
    return (logits, value)


# speedup vs baseline: 1.0106x; 1.0106x over previous
import functools

import jax
import jax.numpy as jnp
from jax import lax
from jax.experimental import pallas as pl
from jax.experimental.pallas import tpu as pltpu
from jax.experimental.pallas import tpu_sc as plsc

N = 10000
E = 320000
H = 128
R = 10
L = 6
G = 64
OUT = 32

NB = 1000
C = 128
NCHUNK = E // C
NRP = 100352
ZW = NRP // 16
NPT = 624

_mesh = plsc.VectorSubcoreMesh(core_axis_name="c", subcore_axis_name="s")


def _count_body(dst_hbm, et_hbm, cnt_out, cnt_sh, dstv, etv, idxv, ones, zbuf, sem):
    cid = lax.axis_index("c")
    sid = lax.axis_index("s")
    wid = cid * 16 + sid

    def _zb(i, _):
        zbuf[pl.ds(i * 16, 16)] = jnp.zeros((16,), jnp.float32)
        return 0

    lax.fori_loop(0, ZW // 16, _zb, 0)
    for k in range(C // 16):
        ones[pl.ds(k * 16, 16)] = jnp.ones((16,), jnp.float32)
    pltpu.sync_copy(zbuf, cnt_sh.at[pl.ds(sid * ZW, ZW)])
    plsc.subcore_barrier()

    ebase = (78 * wid + jnp.minimum(wid, 4)) * C
    extra = (wid < 4).astype(jnp.int32)
    nch = 78 + extra
    pltpu.sync_copy(dst_hbm.at[pl.ds(ebase, 78 * C)], dstv.at[pl.ds(0, 78 * C)])
    pltpu.sync_copy(et_hbm.at[pl.ds(ebase, 78 * C)], etv.at[pl.ds(0, 78 * C)])

    @pl.when(wid < 4)
    def _():
        pltpu.sync_copy(dst_hbm.at[pl.ds(ebase + 78 * C, C)],
                        dstv.at[pl.ds(78 * C, C)])
        pltpu.sync_copy(et_hbm.at[pl.ds(ebase + 78 * C, C)],
                        etv.at[pl.ds(78 * C, C)])

    def _idx(t, _):
        sl = pl.ds(t * 16, 16)
        idxv[sl] = dstv[sl] * R + etv[sl]
        return 0

    lax.fori_loop(0, nch * (C // 16), _idx, 0)

    def _fire(c):
        pltpu.async_copy(ones, cnt_sh.at[idxv.at[pl.ds(c * C, C)]], sem,
                         add=True)

    def _drain(c):
        pltpu.make_async_copy(ones, cnt_sh.at[idxv.at[pl.ds(c * C, C)]],
                              sem).wait()

    def _sc(c, _):
        _fire(c)

        @pl.when(c >= 13)
        def _():
            _drain(c - 13)

        return 0

    lax.fori_loop(0, nch, _sc, 0)

    def _dr(c, _):
        _drain(nch - 13 + c)
        return 0

    lax.fori_loop(0, 13, _dr, 0)
    plsc.subcore_barrier()
    pltpu.sync_copy(cnt_sh.at[pl.ds(sid * ZW, ZW)],
                    cnt_out.at[pl.ds(cid * NRP + sid * ZW, ZW)])


_count_edges = functools.partial(
    pl.kernel,
    out_type=jax.ShapeDtypeStruct((2 * NRP,), jnp.float32),
    mesh=_mesh,
    scratch_types=[
        pltpu.VMEM_SHARED((NRP,), jnp.float32),
        pltpu.VMEM((79 * C,), jnp.int32),
        pltpu.VMEM((79 * C,), jnp.int32),
        pltpu.VMEM((79 * C,), jnp.int32),
        pltpu.VMEM((C,), jnp.float32),
        pltpu.VMEM((ZW,), jnp.float32),
        pltpu.SemaphoreType.DMA,
    ],
)(_count_body)


def _weights_body(src_hbm, dst_hbm, et_hbm, inv_hbm, w_out, gidx_out,
                  inv_v, srcb, dstb, etb, wb, gb):
    cid = lax.axis_index("c")
    sid = lax.axis_index("s")
    wid = cid * 16 + sid
    pltpu.sync_copy(inv_hbm, inv_v)

    ebase = (78 * wid + jnp.minimum(wid, 4)) * C

    def _blk(blk, nw):
        pltpu.sync_copy(src_hbm.at[pl.ds(blk, nw)], srcb.at[pl.ds(0, nw)])
        pltpu.sync_copy(dst_hbm.at[pl.ds(blk, nw)], dstb.at[pl.ds(0, nw)])
        pltpu.sync_copy(et_hbm.at[pl.ds(blk, nw)], etb.at[pl.ds(0, nw)])

        def _t(t, _):
            sl = pl.ds(t * 16, 16)
            sv = srcb[sl]
            dv = dstb[sl]
            ev = etb[sl]
            wb[sl] = plsc.load_gather(inv_v, [dv * R + ev])
            gb[sl] = ev * N + sv
            return 0

        lax.fori_loop(0, nw // 16, _t, 0)
        pltpu.sync_copy(wb.at[pl.ds(0, nw)], w_out.at[pl.ds(blk, nw)])
        pltpu.sync_copy(gb.at[pl.ds(0, nw)], gidx_out.at[pl.ds(blk, nw)])

    for b in range(3):
        _blk(ebase + b * BW, BW)

    @pl.when(wid < 4)
    def _():
        _blk(ebase + 3 * BW, C)


_edge_weights = functools.partial(
    pl.kernel,
    out_type=(jax.ShapeDtypeStruct((E,), jnp.float32),
              jax.ShapeDtypeStruct((E,), jnp.int32)),
    mesh=_mesh,
    scratch_types=[
        pltpu.VMEM((NRP,), jnp.float32),
        pltpu.VMEM((26 * C,), jnp.int32),
        pltpu.VMEM((26 * C,), jnp.int32),
        pltpu.VMEM((26 * C,), jnp.int32),
        pltpu.VMEM((26 * C,), jnp.float32),
        pltpu.VMEM((26 * C,), jnp.int32),
    ],
    compiler_params=pltpu.CompilerParams(needs_layout_passes=False),
)(_weights_body)


NPB = 26
BW = NPB * C


def _scatter_body(hr, gidx_hbm, dst_hbm, w_hbm, accs,
                  acc_sh, idxb, dstb, wb, rows0, rows1,
                  sg0, sg1, ss0, ss1):
    cid = lax.axis_index("c")
    sid = lax.axis_index("s")
    wid = cid * 16 + sid

    def _zb(i, _):
        for k in range(8):
            rows0[i, pl.ds(k * 16, 16)] = jnp.zeros((16,), jnp.float32)
        return 0

    lax.fori_loop(0, C, _zb, 0)
    r0 = sid * NPT
    for m in range(4):
        pltpu.sync_copy(rows0, acc_sh.at[pl.ds(r0 + m * C, C)])
    pltpu.sync_copy(rows0.at[pl.ds(0, NPT - 4 * C)],
                    acc_sh.at[pl.ds(r0 + 4 * C, NPT - 4 * C)])

    @pl.when(sid == 15)
    def _():
        pltpu.sync_copy(rows0.at[pl.ds(0, 16)],
                        acc_sh.at[pl.ds(16 * NPT, 16)])

    plsc.subcore_barrier()

    def _scale(rows, woff):
        def _g(g, _):
            wrow = wb[pl.ds(woff + g * 16, 16)]
            for lane in range(16):
                wv = wrow[lane]
                i = g * 16 + lane
                for k in range(8):
                    sl = pl.ds(k * 16, 16)
                    rows[i, sl] = rows[i, sl] * wv
            return 0

        lax.fori_loop(0, C // 16, _g, 0)

    def _gather(c, rows, sem):
        return pltpu.make_async_copy(hr.at[idxb.at[pl.ds(c * C, C)]],
                                     rows, sem)

    def _scatter(c, rows, sem):
        return pltpu.async_copy(rows, acc_sh.at[dstb.at[pl.ds(c * C, C)]],
                                sem, add=True)

    def _scatter_wait(c, rows, sem):
        pltpu.make_async_copy(rows, acc_sh.at[dstb.at[pl.ds(c * C, C)]],
                              sem).wait()

    ebase = (78 * wid + jnp.minimum(wid, 4)) * C

    for b in range(3):
        blk = ebase + b * BW
        pltpu.sync_copy(gidx_hbm.at[pl.ds(blk, BW)], idxb)
        pltpu.sync_copy(dst_hbm.at[pl.ds(blk, BW)], dstb)
        pltpu.sync_copy(w_hbm.at[pl.ds(blk, BW)], wb)
        _gather(0, rows0, sg0).start()

        def _pair(p, _):
            c0 = 2 * p
            c1 = c0 + 1

            @pl.when(p > 0)
            def _():
                _scatter_wait(c0 - 1, rows1, ss1)

            _gather(c1, rows1, sg1).start()
            _gather(c0, rows0, sg0).wait()
            _scale(rows0, c0 * C)
            _scatter(c0, rows0, ss0)
            _gather(c1, rows1, sg1).wait()
            _scale(rows1, c1 * C)
            _scatter(c1, rows1, ss1)
            _scatter_wait(c0, rows0, ss0)

            @pl.when(p < NPB // 2 - 1)
            def _():
                _gather(c0 + 2, rows0, sg0).start()

            return 0

        lax.fori_loop(0, NPB // 2, _pair, 0)
        _scatter_wait(NPB - 1, rows1, ss1)

    @pl.when(wid < 4)
    def _():
        blk = ebase + 3 * BW
        pltpu.sync_copy(gidx_hbm.at[pl.ds(blk, C)], idxb.at[pl.ds(0, C)])
        pltpu.sync_copy(dst_hbm.at[pl.ds(blk, C)], dstb.at[pl.ds(0, C)])
        pltpu.sync_copy(w_hbm.at[pl.ds(blk, C)], wb.at[pl.ds(0, C)])
        _gather(0, rows0, sg0).start()
        _gather(0, rows0, sg0).wait()
        _scale(rows0, 0)
        _scatter(0, rows0, ss0)
        _scatter_wait(0, rows0, ss0)

    plsc.subcore_barrier()
    pltpu.sync_copy(acc_sh.at[pl.ds(r0, NPT)], accs.at[cid, pl.ds(r0, NPT)])

    @pl.when(sid == 15)
    def _():
        pltpu.sync_copy(acc_sh.at[pl.ds(16 * NPT, 16)],
                        accs.at[cid, pl.ds(16 * NPT, 16)])


_edge_scatter = functools.partial(
    pl.kernel,
    out_type=jax.ShapeDtypeStruct((2, N, H), jnp.float32),
    mesh=_mesh,
    scratch_types=[
        pltpu.VMEM_SHARED((N, H), jnp.float32),
        pltpu.VMEM((BW,), jnp.int32),
        pltpu.VMEM((BW,), jnp.int32),
        pltpu.VMEM((BW,), jnp.float32),
        pltpu.VMEM((C, H), jnp.float32),
        pltpu.VMEM((C, H), jnp.float32),
        pltpu.SemaphoreType.DMA,
        pltpu.SemaphoreType.DMA,
        pltpu.SemaphoreType.DMA,
        pltpu.SemaphoreType.DMA,
    ],
    compiler_params=pltpu.CompilerParams(needs_layout_passes=False),
)(_scatter_body)


def _inv_body(cnt_ref, inv_ref):
    c = cnt_ref[0] + cnt_ref[1]
    inv_ref[...] = 1.0 / jnp.maximum(c, 1.0)


def _inv_counts(cnt2d):
    return pl.pallas_call(
        _inv_body,
        out_shape=jax.ShapeDtypeStruct((NRP,), jnp.float32),
    )(cnt2d)


def _ln_relu_res(acc0, acc1, root, bcv, gam, bet, hprev):
    s = acc0 + acc1 + root + bcv
    mu = jnp.mean(s, axis=-1, keepdims=True)
    var = jnp.mean((s - mu) ** 2, axis=-1, keepdims=True)
    s = (s - mu) * lax.rsqrt(var + 1e-5) * gam + bet
    return jnp.maximum(s, 0.0) + hprev


def _transforms(h, wrel_ref, wroot_ref, hr_ref, root_ref):
    for r in range(R):
        hr_ref[r] = jnp.dot(h, wrel_ref[r], preferred_element_type=jnp.float32)
    root_ref[...] = jnp.dot(h, wroot_ref[...], preferred_element_type=jnp.float32)


def _dense_in_body(x_ref, wp_ref, bp_ref, wrel_ref, wroot_ref,
                   h_ref, hr_ref, root_ref):
    h = jnp.dot(x_ref[...], wp_ref[...],
                preferred_element_type=jnp.float32) + bp_ref[...]
    h_ref[...] = h
    _transforms(h, wrel_ref, wroot_ref, hr_ref, root_ref)


def _dense_mid_body(acc0_ref, acc1_ref, rootin_ref, hprev_ref, bcv_ref,
                    gam_ref, bet_ref, wrel_ref, wroot_ref,
                    h_ref, hr_ref, root_ref):
    h = _ln_relu_res(acc0_ref[...], acc1_ref[...], rootin_ref[...],
                     bcv_ref[...], gam_ref[...], bet_ref[...], hprev_ref[...])
    h_ref[...] = h
    _transforms(h, wrel_ref, wroot_ref, hr_ref, root_ref)


def _dense_out_body(acc0_ref, acc1_ref, rootin_ref, hprev_ref, bcv_ref,
                    gam_ref, bet_ref, batch_ref,
                    wm1_ref, bm1_ref, wm2_ref, bm2_ref,
                    wv1_ref, bv1_ref, wv2_ref, bv2_ref,
                    logits_ref, value_ref, g_acc):
    i = pl.program_id(0)

    @pl.when(i == 0)
    def _():
        g_acc[...] = jnp.zeros_like(g_acc)

    h = _ln_relu_res(acc0_ref[...], acc1_ref[...], rootin_ref[...],
                     bcv_ref[...], gam_ref[...], bet_ref[...], hprev_ref[...])
    b = batch_ref[...].reshape(NB)
    oh = (lax.broadcasted_iota(jnp.int32, (G, NB), 0) == b[None, :])
    g_acc[...] += jnp.dot(oh.astype(jnp.float32), h,
                          preferred_element_type=jnp.float32)

    @pl.when(i == pl.num_programs(0) - 1)
    def _():
        g = g_acc[...]
        hm = jnp.maximum(jnp.dot(g, wm1_ref[...],
                                 preferred_element_type=jnp.float32)
                         + bm1_ref[...], 0.0)
        logits_ref[...] = jnp.dot(hm, wm2_ref[...],
                                  preferred_element_type=jnp.float32) + bm2_ref[...]
        hv = jnp.maximum(jnp.dot(g, wv1_ref[...],
                                 preferred_element_type=jnp.float32)
                         + bv1_ref[...], 0.0)
        value_ref[...] = jnp.dot(hv, wv2_ref[...],
                                 preferred_element_type=jnp.float32) + bv2_ref[...]


_row_spec = pl.BlockSpec((NB, H), lambda i: (i, 0))
_vecH_spec = pl.BlockSpec((H,), lambda i: (0,))
_wrel_spec = pl.BlockSpec((R, H, H), lambda i: (0, 0, 0))
_wHH_spec = pl.BlockSpec((H, H), lambda i: (0, 0))
_hr_spec = pl.BlockSpec((R, NB, H), lambda i: (0, i, 0))

_h_hr_root_shapes = (jax.ShapeDtypeStruct((N, H), jnp.float32),
                     jax.ShapeDtypeStruct((R, N, H), jnp.float32),
                     jax.ShapeDtypeStruct((N, H), jnp.float32))


def _dense_in(x, Wp, bp, Wrel0, Wroot0):
    return pl.pallas_call(
        _dense_in_body,
        grid=(N // NB,),
        in_specs=[_row_spec, _wHH_spec, _vecH_spec, _wrel_spec, _wHH_spec],
        out_specs=[_row_spec, _hr_spec, _row_spec],
        out_shape=_h_hr_root_shapes,
    )(x, Wp, bp, Wrel0, Wroot0)


def _dense_mid(acc0, acc1, root, hprev, bcv, gam, bet, Wrel_l, Wroot_l):
    return pl.pallas_call(
        _dense_mid_body,
        grid=(N // NB,),
        in_specs=[_row_spec, _row_spec, _row_spec, _row_spec,
                  _vecH_spec, _vecH_spec, _vecH_spec, _wrel_spec, _wHH_spec],
        out_specs=[_row_spec, _hr_spec, _row_spec],
        out_shape=_h_hr_root_shapes,
    )(acc0, acc1, root, hprev, bcv, gam, bet, Wrel_l, Wroot_l)


def _dense_out(acc0, acc1, root, hprev, bcv, gam, bet, batch3d,
               Wm1, bm1, Wm2, bm2, Wv1, bv1, Wv2, bv2):
    wH = pl.BlockSpec((H, H), lambda i: (0, 0))
    wO = pl.BlockSpec((H, OUT), lambda i: (0, 0))
    vO = pl.BlockSpec((OUT,), lambda i: (0,))
    out_spec = pl.BlockSpec((G, OUT), lambda i: (0, 0))
    return pl.pallas_call(
        _dense_out_body,
        grid=(N // NB,),
        in_specs=[_row_spec, _row_spec, _row_spec, _row_spec,
                  _vecH_spec, _vecH_spec, _vecH_spec,
                  pl.BlockSpec((1, 1, NB), lambda i: (i, 0, 0)),
                  wH, _vecH_spec, wO, vO, wH, _vecH_spec, wO, vO],
        out_specs=[out_spec, out_spec],
        out_shape=(jax.ShapeDtypeStruct((G, OUT), jnp.float32),
                   jax.ShapeDtypeStruct((G, OUT), jnp.float32)),
        scratch_shapes=[pltpu.VMEM((G, H), jnp.float32)],
    )(acc0, acc1, root, hprev, bcv, gam, bet, batch3d,
      Wm1, bm1, Wm2, bm2, Wv1, bv1, Wv2, bv2)


def kernel(x, edge_index, edge_type, batch, Wp, bp, Wrel, Wroot, bconv,
           gamma, beta, Wm1, bm1, Wm2, bm2, Wv1, bv1, Wv2, bv2):
    src = edge_index[0].astype(jnp.int32)
    dst = edge_index[1].astype(jnp.int32)
    et = edge_type.astype(jnp.int32)
    batch3d = batch.astype(jnp.int32).reshape(N // NB, 1, NB)

    cnt = _count_edges(dst, et)
    inv = _inv_counts(cnt.reshape(2, NRP))
    w, gidx = _edge_weights(src, dst, et, inv)

    h, hr, root = _dense_in(x, Wp, bp, Wrel[0], Wroot[0])
    for l in range(L):
        accs = _edge_scatter(hr.reshape(R * N, H), gidx, dst, w)
        if l < L - 1:
            h, hr, root = _dense_mid(accs[0], accs[1], root, h, bconv[l],
                                     gamma[l], beta[l], Wrel[l + 1],
                                     Wroot[l + 1])
        else:
            logits, value = _dense_out(accs[0], accs[1], root, h, bconv[l],
                                       gamma[l], beta[l], batch3d,
                                       Wm1, bm1, Wm2, bm2, Wv1, bv1, Wv2, bv2)
    return (logits, value)


# speedup vs baseline: 1.0852x; 1.0738x over previous
import functools

import jax
import jax.numpy as jnp
from jax import lax
from jax.experimental import pallas as pl
from jax.experimental.pallas import tpu as pltpu
from jax.experimental.pallas import tpu_sc as plsc

N = 10000
E = 320000
H = 128
R = 10
L = 6
G = 64
OUT = 32

NB = 1000
C = 128
NCHUNK = E // C
NRP = 100352
ZW = NRP // 16
NPT = 624

_mesh = plsc.VectorSubcoreMesh(core_axis_name="c", subcore_axis_name="s")


def _count_body(dst_hbm, et_hbm, cnt_out, cnt_sh, dstv, etv, idxv, ones, zbuf, sem):
    cid = lax.axis_index("c")
    sid = lax.axis_index("s")
    wid = cid * 16 + sid

    def _zb(i, _):
        zbuf[pl.ds(i * 16, 16)] = jnp.zeros((16,), jnp.float32)
        return 0

    lax.fori_loop(0, ZW // 16, _zb, 0)
    for k in range(C // 16):
        ones[pl.ds(k * 16, 16)] = jnp.ones((16,), jnp.float32)
    pltpu.sync_copy(zbuf, cnt_sh.at[pl.ds(sid * ZW, ZW)])
    plsc.subcore_barrier()

    ebase = (78 * wid + jnp.minimum(wid, 4)) * C
    extra = (wid < 4).astype(jnp.int32)
    nch = 78 + extra
    pltpu.sync_copy(dst_hbm.at[pl.ds(ebase, 78 * C)], dstv.at[pl.ds(0, 78 * C)])
    pltpu.sync_copy(et_hbm.at[pl.ds(ebase, 78 * C)], etv.at[pl.ds(0, 78 * C)])

    @pl.when(wid < 4)
    def _():
        pltpu.sync_copy(dst_hbm.at[pl.ds(ebase + 78 * C, C)],
                        dstv.at[pl.ds(78 * C, C)])
        pltpu.sync_copy(et_hbm.at[pl.ds(ebase + 78 * C, C)],
                        etv.at[pl.ds(78 * C, C)])

    def _idx(t, _):
        sl = pl.ds(t * 16, 16)
        idxv[sl] = dstv[sl] * R + etv[sl]
        return 0

    lax.fori_loop(0, nch * (C // 16), _idx, 0)

    def _fire(c):
        pltpu.async_copy(ones, cnt_sh.at[idxv.at[pl.ds(c * C, C)]], sem,
                         add=True)

    def _drain(c):
        pltpu.make_async_copy(ones, cnt_sh.at[idxv.at[pl.ds(c * C, C)]],
                              sem).wait()

    def _sc(c, _):
        _fire(c)

        @pl.when(c >= 13)
        def _():
            _drain(c - 13)

        return 0

    lax.fori_loop(0, nch, _sc, 0)

    def _dr(c, _):
        _drain(nch - 13 + c)
        return 0

    lax.fori_loop(0, 13, _dr, 0)
    plsc.subcore_barrier()
    pltpu.sync_copy(cnt_sh.at[pl.ds(sid * ZW, ZW)],
                    cnt_out.at[pl.ds(cid * NRP + sid * ZW, ZW)])


_count_edges = functools.partial(
    pl.kernel,
    out_type=jax.ShapeDtypeStruct((2 * NRP,), jnp.float32),
    mesh=_mesh,
    scratch_types=[
        pltpu.VMEM_SHARED((NRP,), jnp.float32),
        pltpu.VMEM((79 * C,), jnp.int32),
        pltpu.VMEM((79 * C,), jnp.int32),
        pltpu.VMEM((79 * C,), jnp.int32),
        pltpu.VMEM((C,), jnp.float32),
        pltpu.VMEM((ZW,), jnp.float32),
        pltpu.SemaphoreType.DMA,
    ],
)(_count_body)


def _weights_body(src_hbm, dst_hbm, et_hbm, inv_hbm, w_out, gidx_out,
                  inv_v, srcb, dstb, etb, wb, gb):
    cid = lax.axis_index("c")
    sid = lax.axis_index("s")
    wid = cid * 16 + sid
    pltpu.sync_copy(inv_hbm, inv_v)

    ebase = (78 * wid + jnp.minimum(wid, 4)) * C

    def _blk(blk, nw):
        pltpu.sync_copy(src_hbm.at[pl.ds(blk, nw)], srcb.at[pl.ds(0, nw)])
        pltpu.sync_copy(dst_hbm.at[pl.ds(blk, nw)], dstb.at[pl.ds(0, nw)])
        pltpu.sync_copy(et_hbm.at[pl.ds(blk, nw)], etb.at[pl.ds(0, nw)])

        def _t(t, _):
            sl = pl.ds(t * 16, 16)
            sv = srcb[sl]
            dv = dstb[sl]
            ev = etb[sl]
            wb[sl] = plsc.load_gather(inv_v, [dv * R + ev])
            gb[sl] = ev * N + sv
            return 0

        lax.fori_loop(0, nw // 16, _t, 0)
        pltpu.sync_copy(wb.at[pl.ds(0, nw)], w_out.at[pl.ds(blk, nw)])
        pltpu.sync_copy(gb.at[pl.ds(0, nw)], gidx_out.at[pl.ds(blk, nw)])

    for b in range(3):
        _blk(ebase + b * BW, BW)

    @pl.when(wid < 4)
    def _():
        _blk(ebase + 3 * BW, C)


_edge_weights = functools.partial(
    pl.kernel,
    out_type=(jax.ShapeDtypeStruct((E,), jnp.float32),
              jax.ShapeDtypeStruct((E,), jnp.int32)),
    mesh=_mesh,
    scratch_types=[
        pltpu.VMEM((NRP,), jnp.float32),
        pltpu.VMEM((26 * C,), jnp.int32),
        pltpu.VMEM((26 * C,), jnp.int32),
        pltpu.VMEM((26 * C,), jnp.int32),
        pltpu.VMEM((26 * C,), jnp.float32),
        pltpu.VMEM((26 * C,), jnp.int32),
    ],
    compiler_params=pltpu.CompilerParams(needs_layout_passes=False),
)(_weights_body)


NPB = 26
BW = NPB * C


def _scatter_body(hr, gidx_hbm, dst_hbm, w_hbm, accs,
                  acc_sh, idxb, dstb, wb, rows0, rows1,
                  sg0, sg1, ss0, ss1):
    cid = lax.axis_index("c")
    sid = lax.axis_index("s")
    wid = cid * 16 + sid

    def _zb(i, _):
        for k in range(8):
            rows0[i, pl.ds(k * 16, 16)] = jnp.zeros((16,), jnp.float32)
        return 0

    lax.fori_loop(0, C, _zb, 0)
    r0 = sid * NPT
    for m in range(4):
        pltpu.sync_copy(rows0, acc_sh.at[pl.ds(r0 + m * C, C)])
    pltpu.sync_copy(rows0.at[pl.ds(0, NPT - 4 * C)],
                    acc_sh.at[pl.ds(r0 + 4 * C, NPT - 4 * C)])

    @pl.when(sid == 15)
    def _():
        pltpu.sync_copy(rows0.at[pl.ds(0, 16)],
                        acc_sh.at[pl.ds(16 * NPT, 16)])

    plsc.subcore_barrier()

    def _scale(rows, woff):
        def _g(g, _):
            wrow = wb[pl.ds(woff + g * 16, 16)]
            for lane in range(16):
                wv = wrow[lane]
                i = g * 16 + lane
                for k in range(8):
                    sl = pl.ds(k * 16, 16)
                    rows[i, sl] = rows[i, sl] * wv
            return 0

        lax.fori_loop(0, C // 16, _g, 0)

    def _gather(c, rows, sem):
        return pltpu.make_async_copy(hr.at[idxb.at[pl.ds(c * C, C)]],
                                     rows, sem)

    def _scatter(c, rows, sem):
        return pltpu.async_copy(rows, acc_sh.at[dstb.at[pl.ds(c * C, C)]],
                                sem, add=True)

    def _scatter_wait(c, rows, sem):
        pltpu.make_async_copy(rows, acc_sh.at[dstb.at[pl.ds(c * C, C)]],
                              sem).wait()

    ebase = (78 * wid + jnp.minimum(wid, 4)) * C

    for b in range(3):
        blk = ebase + b * BW
        pltpu.sync_copy(gidx_hbm.at[pl.ds(blk, BW)], idxb)
        pltpu.sync_copy(dst_hbm.at[pl.ds(blk, BW)], dstb)
        pltpu.sync_copy(w_hbm.at[pl.ds(blk, BW)], wb)
        _gather(0, rows0, sg0).start()

        def _pair(p, _):
            c0 = 2 * p
            c1 = c0 + 1
            _gather(c0, rows0, sg0).wait()

            @pl.when(p > 0)
            def _():
                _scatter_wait(c0 - 1, rows1, ss1)

            _gather(c1, rows1, sg1).start()
            _scale(rows0, c0 * C)
            _scatter(c0, rows0, ss0)
            _gather(c1, rows1, sg1).wait()
            _scatter_wait(c0, rows0, ss0)

            @pl.when(p < NPB // 2 - 1)
            def _():
                _gather(c0 + 2, rows0, sg0).start()

            _scale(rows1, c1 * C)
            _scatter(c1, rows1, ss1)
            return 0

        lax.fori_loop(0, NPB // 2, _pair, 0)
        _scatter_wait(NPB - 1, rows1, ss1)

    @pl.when(wid < 4)
    def _():
        blk = ebase + 3 * BW
        pltpu.sync_copy(gidx_hbm.at[pl.ds(blk, C)], idxb.at[pl.ds(0, C)])
        pltpu.sync_copy(dst_hbm.at[pl.ds(blk, C)], dstb.at[pl.ds(0, C)])
        pltpu.sync_copy(w_hbm.at[pl.ds(blk, C)], wb.at[pl.ds(0, C)])
        _gather(0, rows0, sg0).start()
        _gather(0, rows0, sg0).wait()
        _scale(rows0, 0)
        _scatter(0, rows0, ss0)
        _scatter_wait(0, rows0, ss0)

    plsc.subcore_barrier()
    pltpu.sync_copy(acc_sh.at[pl.ds(r0, NPT)], accs.at[cid, pl.ds(r0, NPT)])

    @pl.when(sid == 15)
    def _():
        pltpu.sync_copy(acc_sh.at[pl.ds(16 * NPT, 16)],
                        accs.at[cid, pl.ds(16 * NPT, 16)])


_edge_scatter = functools.partial(
    pl.kernel,
    out_type=jax.ShapeDtypeStruct((2, N, H), jnp.float32),
    mesh=_mesh,
    scratch_types=[
        pltpu.VMEM_SHARED((N, H), jnp.float32),
        pltpu.VMEM((BW,), jnp.int32),
        pltpu.VMEM((BW,), jnp.int32),
        pltpu.VMEM((BW,), jnp.float32),
        pltpu.VMEM((C, H), jnp.float32),
        pltpu.VMEM((C, H), jnp.float32),
        pltpu.SemaphoreType.DMA,
        pltpu.SemaphoreType.DMA,
        pltpu.SemaphoreType.DMA,
        pltpu.SemaphoreType.DMA,
    ],
    compiler_params=pltpu.CompilerParams(needs_layout_passes=False),
)(_scatter_body)


def _inv_body(cnt_ref, inv_ref):
    c = cnt_ref[0] + cnt_ref[1]
    inv_ref[...] = 1.0 / jnp.maximum(c, 1.0)


def _inv_counts(cnt2d):
    return pl.pallas_call(
        _inv_body,
        out_shape=jax.ShapeDtypeStruct((NRP,), jnp.float32),
    )(cnt2d)


def _ln_relu_res(acc0, acc1, root, bcv, gam, bet, hprev):
    s = acc0 + acc1 + root + bcv
    mu = jnp.mean(s, axis=-1, keepdims=True)
    var = jnp.mean((s - mu) ** 2, axis=-1, keepdims=True)
    s = (s - mu) * lax.rsqrt(var + 1e-5) * gam + bet
    return jnp.maximum(s, 0.0) + hprev


def _transforms(h, wrel_ref, wroot_ref, hr_ref, root_ref):
    for r in range(R):
        hr_ref[r] = jnp.dot(h, wrel_ref[r], preferred_element_type=jnp.float32)
    root_ref[...] = jnp.dot(h, wroot_ref[...], preferred_element_type=jnp.float32)


def _dense_in_body(x_ref, wp_ref, bp_ref, wrel_ref, wroot_ref,
                   h_ref, hr_ref, root_ref):
    h = jnp.dot(x_ref[...], wp_ref[...],
                preferred_element_type=jnp.float32) + bp_ref[...]
    h_ref[...] = h
    _transforms(h, wrel_ref, wroot_ref, hr_ref, root_ref)


def _dense_mid_body(acc0_ref, acc1_ref, rootin_ref, hprev_ref, bcv_ref,
                    gam_ref, bet_ref, wrel_ref, wroot_ref,
                    h_ref, hr_ref, root_ref):
    h = _ln_relu_res(acc0_ref[...], acc1_ref[...], rootin_ref[...],
                     bcv_ref[...], gam_ref[...], bet_ref[...], hprev_ref[...])
    h_ref[...] = h
    _transforms(h, wrel_ref, wroot_ref, hr_ref, root_ref)


def _dense_out_body(acc0_ref, acc1_ref, rootin_ref, hprev_ref, bcv_ref,
                    gam_ref, bet_ref, batch_ref,
                    wm1_ref, bm1_ref, wm2_ref, bm2_ref,
                    wv1_ref, bv1_ref, wv2_ref, bv2_ref,
                    logits_ref, value_ref, g_acc):
    i = pl.program_id(0)

    @pl.when(i == 0)
    def _():
        g_acc[...] = jnp.zeros_like(g_acc)

    h = _ln_relu_res(acc0_ref[...], acc1_ref[...], rootin_ref[...],
                     bcv_ref[...], gam_ref[...], bet_ref[...], hprev_ref[...])
    b = batch_ref[...].reshape(NB)
    oh = (lax.broadcasted_iota(jnp.int32, (G, NB), 0) == b[None, :])
    g_acc[...] += jnp.dot(oh.astype(jnp.float32), h,
                          preferred_element_type=jnp.float32)

    @pl.when(i == pl.num_programs(0) - 1)
    def _():
        g = g_acc[...]
        hm = jnp.maximum(jnp.dot(g, wm1_ref[...],
                                 preferred_element_type=jnp.float32)
                         + bm1_ref[...], 0.0)
        logits_ref[...] = jnp.dot(hm, wm2_ref[...],
                                  preferred_element_type=jnp.float32) + bm2_ref[...]
        hv = jnp.maximum(jnp.dot(g, wv1_ref[...],
                                 preferred_element_type=jnp.float32)
                         + bv1_ref[...], 0.0)
        value_ref[...] = jnp.dot(hv, wv2_ref[...],
                                 preferred_element_type=jnp.float32) + bv2_ref[...]


_row_spec = pl.BlockSpec((NB, H), lambda i: (i, 0))
_vecH_spec = pl.BlockSpec((H,), lambda i: (0,))
_wrel_spec = pl.BlockSpec((R, H, H), lambda i: (0, 0, 0))
_wHH_spec = pl.BlockSpec((H, H), lambda i: (0, 0))
_hr_spec = pl.BlockSpec((R, NB, H), lambda i: (0, i, 0))

_h_hr_root_shapes = (jax.ShapeDtypeStruct((N, H), jnp.float32),
                     jax.ShapeDtypeStruct((R, N, H), jnp.float32),
                     jax.ShapeDtypeStruct((N, H), jnp.float32))


def _dense_in(x, Wp, bp, Wrel0, Wroot0):
    return pl.pallas_call(
        _dense_in_body,
        grid=(N // NB,),
        in_specs=[_row_spec, _wHH_spec, _vecH_spec, _wrel_spec, _wHH_spec],
        out_specs=[_row_spec, _hr_spec, _row_spec],
        out_shape=_h_hr_root_shapes,
    )(x, Wp, bp, Wrel0, Wroot0)


def _dense_mid(acc0, acc1, root, hprev, bcv, gam, bet, Wrel_l, Wroot_l):
    return pl.pallas_call(
        _dense_mid_body,
        grid=(N // NB,),
        in_specs=[_row_spec, _row_spec, _row_spec, _row_spec,
                  _vecH_spec, _vecH_spec, _vecH_spec, _wrel_spec, _wHH_spec],
        out_specs=[_row_spec, _hr_spec, _row_spec],
        out_shape=_h_hr_root_shapes,
    )(acc0, acc1, root, hprev, bcv, gam, bet, Wrel_l, Wroot_l)


def _dense_out(acc0, acc1, root, hprev, bcv, gam, bet, batch3d,
               Wm1, bm1, Wm2, bm2, Wv1, bv1, Wv2, bv2):
    wH = pl.BlockSpec((H, H), lambda i: (0, 0))
    wO = pl.BlockSpec((H, OUT), lambda i: (0, 0))
    vO = pl.BlockSpec((OUT,), lambda i: (0,))
    out_spec = pl.BlockSpec((G, OUT), lambda i: (0, 0))
    return pl.pallas_call(
        _dense_out_body,
        grid=(N // NB,),
        in_specs=[_row_spec, _row_spec, _row_spec, _row_spec,
                  _vecH_spec, _vecH_spec, _vecH_spec,
                  pl.BlockSpec((1, 1, NB), lambda i: (i, 0, 0)),
                  wH, _vecH_spec, wO, vO, wH, _vecH_spec, wO, vO],
        out_specs=[out_spec, out_spec],
        out_shape=(jax.ShapeDtypeStruct((G, OUT), jnp.float32),
                   jax.ShapeDtypeStruct((G, OUT), jnp.float32)),
        scratch_shapes=[pltpu.VMEM((G, H), jnp.float32)],
    )(acc0, acc1, root, hprev, bcv, gam, bet, batch3d,
      Wm1, bm1, Wm2, bm2, Wv1, bv1, Wv2, bv2)


def kernel(x, edge_index, edge_type, batch, Wp, bp, Wrel, Wroot, bconv,
           gamma, beta, Wm1, bm1, Wm2, bm2, Wv1, bv1, Wv2, bv2):
    src = edge_index[0].astype(jnp.int32)
    dst = edge_index[1].astype(jnp.int32)
    et = edge_type.astype(jnp.int32)
    batch3d = batch.astype(jnp.int32).reshape(N // NB, 1, NB)

    cnt = _count_edges(dst, et)
    inv = _inv_counts(cnt.reshape(2, NRP))
    w, gidx = _edge_weights(src, dst, et, inv)

    h, hr, root = _dense_in(x, Wp, bp, Wrel[0], Wroot[0])
    for l in range(L):
        accs = _edge_scatter(hr.reshape(R * N, H), gidx, dst, w)
        if l < L - 1:
            h, hr, root = _dense_mid(accs[0], accs[1], root, h, bconv[l],
                                     gamma[l], beta[l], Wrel[l + 1],
                                     Wroot[l + 1])
        else:
            logits, value = _dense_out(accs[0], accs[1], root, h, bconv[l],
                                       gamma[l], beta[l], batch3d,
                                       Wm1, bm1, Wm2, bm2, Wv1, bv1, Wv2, bv2)
    return (logits, value)


# speedup vs baseline: 1.0886x; 1.0032x over previous
import functools

import jax
import jax.numpy as jnp
from jax import lax
from jax.experimental import pallas as pl
from jax.experimental.pallas import tpu as pltpu
from jax.experimental.pallas import tpu_sc as plsc

N = 10000
E = 320000
H = 128
R = 10
L = 6
G = 64
OUT = 32

NB = 1000
C = 128
NCHUNK = E // C
NRP = 100352
ZW = NRP // 16
NPT = 624

_mesh = plsc.VectorSubcoreMesh(core_axis_name="c", subcore_axis_name="s")


def _count_body(dst_hbm, et_hbm, cnt_out, cnt_sh, dstv, etv, idxv, ones, zbuf, sem):
    cid = lax.axis_index("c")
    sid = lax.axis_index("s")
    wid = cid * 16 + sid

    def _zb(i, _):
        zbuf[pl.ds(i * 16, 16)] = jnp.zeros((16,), jnp.float32)
        return 0

    lax.fori_loop(0, ZW // 16, _zb, 0)
    for k in range(C // 16):
        ones[pl.ds(k * 16, 16)] = jnp.ones((16,), jnp.float32)
    pltpu.sync_copy(zbuf, cnt_sh.at[pl.ds(sid * ZW, ZW)])
    plsc.subcore_barrier()

    ebase = (78 * wid + jnp.minimum(wid, 4)) * C
    extra = (wid < 4).astype(jnp.int32)
    nch = 78 + extra
    pltpu.sync_copy(dst_hbm.at[pl.ds(ebase, 78 * C)], dstv.at[pl.ds(0, 78 * C)])
    pltpu.sync_copy(et_hbm.at[pl.ds(ebase, 78 * C)], etv.at[pl.ds(0, 78 * C)])

    @pl.when(wid < 4)
    def _():
        pltpu.sync_copy(dst_hbm.at[pl.ds(ebase + 78 * C, C)],
                        dstv.at[pl.ds(78 * C, C)])
        pltpu.sync_copy(et_hbm.at[pl.ds(ebase + 78 * C, C)],
                        etv.at[pl.ds(78 * C, C)])

    def _idx(t, _):
        sl = pl.ds(t * 16, 16)
        idxv[sl] = dstv[sl] * R + etv[sl]
        return 0

    lax.fori_loop(0, nch * (C // 16), _idx, 0)

    def _fire(c):
        pltpu.async_copy(ones, cnt_sh.at[idxv.at[pl.ds(c * C, C)]], sem,
                         add=True)

    def _drain(c):
        pltpu.make_async_copy(ones, cnt_sh.at[idxv.at[pl.ds(c * C, C)]],
                              sem).wait()

    def _sc(c, _):
        _fire(c)

        @pl.when(c >= 13)
        def _():
            _drain(c - 13)

        return 0

    lax.fori_loop(0, nch, _sc, 0)

    def _dr(c, _):
        _drain(nch - 13 + c)
        return 0

    lax.fori_loop(0, 13, _dr, 0)
    plsc.subcore_barrier()
    pltpu.sync_copy(cnt_sh.at[pl.ds(sid * ZW, ZW)],
                    cnt_out.at[pl.ds(cid * NRP + sid * ZW, ZW)])


_count_edges = functools.partial(
    pl.kernel,
    out_type=jax.ShapeDtypeStruct((2 * NRP,), jnp.float32),
    mesh=_mesh,
    scratch_types=[
        pltpu.VMEM_SHARED((NRP,), jnp.float32),
        pltpu.VMEM((79 * C,), jnp.int32),
        pltpu.VMEM((79 * C,), jnp.int32),
        pltpu.VMEM((79 * C,), jnp.int32),
        pltpu.VMEM((C,), jnp.float32),
        pltpu.VMEM((ZW,), jnp.float32),
        pltpu.SemaphoreType.DMA,
    ],
)(_count_body)


def _weights_body(src_hbm, dst_hbm, et_hbm, inv_hbm, w_out, gidx_out,
                  inv_v, srcb, dstb, etb, wb, gb):
    cid = lax.axis_index("c")
    sid = lax.axis_index("s")
    wid = cid * 16 + sid
    pltpu.sync_copy(inv_hbm, inv_v)

    ebase = (78 * wid + jnp.minimum(wid, 4)) * C

    def _blk(blk, nw):
        pltpu.sync_copy(src_hbm.at[pl.ds(blk, nw)], srcb.at[pl.ds(0, nw)])
        pltpu.sync_copy(dst_hbm.at[pl.ds(blk, nw)], dstb.at[pl.ds(0, nw)])
        pltpu.sync_copy(et_hbm.at[pl.ds(blk, nw)], etb.at[pl.ds(0, nw)])

        def _t(t, _):
            sl = pl.ds(t * 16, 16)
            sv = srcb[sl]
            dv = dstb[sl]
            ev = etb[sl]
            wb[sl] = plsc.load_gather(inv_v, [dv * R + ev])
            gb[sl] = ev * N + sv
            return 0

        lax.fori_loop(0, nw // 16, _t, 0)
        pltpu.sync_copy(wb.at[pl.ds(0, nw)], w_out.at[pl.ds(blk, nw)])
        pltpu.sync_copy(gb.at[pl.ds(0, nw)], gidx_out.at[pl.ds(blk, nw)])

    for b in range(3):
        _blk(ebase + b * BW, BW)

    @pl.when(wid < 4)
    def _():
        _blk(ebase + 3 * BW, C)


_edge_weights = functools.partial(
    pl.kernel,
    out_type=(jax.ShapeDtypeStruct((E,), jnp.float32),
              jax.ShapeDtypeStruct((E,), jnp.int32)),
    mesh=_mesh,
    scratch_types=[
        pltpu.VMEM((NRP,), jnp.float32),
        pltpu.VMEM((26 * C,), jnp.int32),
        pltpu.VMEM((26 * C,), jnp.int32),
        pltpu.VMEM((26 * C,), jnp.int32),
        pltpu.VMEM((26 * C,), jnp.float32),
        pltpu.VMEM((26 * C,), jnp.int32),
    ],
    compiler_params=pltpu.CompilerParams(needs_layout_passes=False),
)(_weights_body)


NPB = 26
BW = NPB * C


def _scatter_body(hr, gidx_hbm, dst_hbm, w_hbm, accs,
                  acc_sh, idxb, dstb, wb, rows0, rows1,
                  sg0, sg1, ss0, ss1):
    cid = lax.axis_index("c")
    sid = lax.axis_index("s")
    wid = cid * 16 + sid

    def _zb(i, _):
        for k in range(8):
            rows0[i, pl.ds(k * 16, 16)] = jnp.zeros((16,), jnp.float32)
        return 0

    lax.fori_loop(0, C, _zb, 0)
    r0 = sid * NPT
    for m in range(4):
        pltpu.sync_copy(rows0, acc_sh.at[pl.ds(r0 + m * C, C)])
    pltpu.sync_copy(rows0.at[pl.ds(0, NPT - 4 * C)],
                    acc_sh.at[pl.ds(r0 + 4 * C, NPT - 4 * C)])

    @pl.when(sid == 15)
    def _():
        pltpu.sync_copy(rows0.at[pl.ds(0, 16)],
                        acc_sh.at[pl.ds(16 * NPT, 16)])

    plsc.subcore_barrier()

    def _scale(rows, woff):
        def _g(g, _):
            wrow = wb[pl.ds(woff + g * 16, 16)]
            for lane in range(16):
                wv = wrow[lane]
                i = g * 16 + lane
                for k in range(8):
                    sl = pl.ds(k * 16, 16)
                    rows[i, sl] = rows[i, sl] * wv
            return 0

        lax.fori_loop(0, C // 16, _g, 0)

    def _gather(c, rows, sem):
        return pltpu.make_async_copy(hr.at[idxb.at[pl.ds(c * C, C)]],
                                     rows, sem)

    def _scatter(c, rows, sem):
        return pltpu.async_copy(rows, acc_sh.at[dstb.at[pl.ds(c * C, C)]],
                                sem, add=True)

    def _scatter_wait(c, rows, sem):
        pltpu.make_async_copy(rows, acc_sh.at[dstb.at[pl.ds(c * C, C)]],
                              sem).wait()

    ebase = (78 * wid + jnp.minimum(wid, 4)) * C

    for b in range(3):
        blk = ebase + b * BW
        pltpu.sync_copy(gidx_hbm.at[pl.ds(blk, BW)], idxb)
        pltpu.sync_copy(dst_hbm.at[pl.ds(blk, BW)], dstb)
        pltpu.sync_copy(w_hbm.at[pl.ds(blk, BW)], wb)
        _gather(0, rows0, sg0).start()

        def _pair(p, _):
            c0 = 2 * p
            c1 = c0 + 1
            _gather(c0, rows0, sg0).wait()

            @pl.when(p > 0)
            def _():
                _scatter_wait(c0 - 1, rows1, ss1)

            _gather(c1, rows1, sg1).start()
            _scale(rows0, c0 * C)
            _scatter(c0, rows0, ss0)
            _gather(c1, rows1, sg1).wait()
            _scatter_wait(c0, rows0, ss0)

            @pl.when(p < NPB // 2 - 1)
            def _():
                _gather(c0 + 2, rows0, sg0).start()

            _scale(rows1, c1 * C)
            _scatter(c1, rows1, ss1)
            return 0

        lax.fori_loop(0, NPB // 2, _pair, 0)
        _scatter_wait(NPB - 1, rows1, ss1)

    @pl.when(wid < 4)
    def _():
        blk = ebase + 3 * BW
        pltpu.sync_copy(gidx_hbm.at[pl.ds(blk, C)], idxb.at[pl.ds(0, C)])
        pltpu.sync_copy(dst_hbm.at[pl.ds(blk, C)], dstb.at[pl.ds(0, C)])
        pltpu.sync_copy(w_hbm.at[pl.ds(blk, C)], wb.at[pl.ds(0, C)])
        _gather(0, rows0, sg0).start()
        _gather(0, rows0, sg0).wait()
        _scale(rows0, 0)
        _scatter(0, rows0, ss0)
        _scatter_wait(0, rows0, ss0)

    plsc.subcore_barrier()
    pltpu.sync_copy(acc_sh.at[pl.ds(r0, NPT)], accs.at[cid, pl.ds(r0, NPT)])

    @pl.when(sid == 15)
    def _():
        pltpu.sync_copy(acc_sh.at[pl.ds(16 * NPT, 16)],
                        accs.at[cid, pl.ds(16 * NPT, 16)])


_edge_scatter = functools.partial(
    pl.kernel,
    out_type=jax.ShapeDtypeStruct((2, N, H), jnp.float32),
    mesh=_mesh,
    scratch_types=[
        pltpu.VMEM_SHARED((N, H), jnp.float32),
        pltpu.VMEM((BW,), jnp.int32),
        pltpu.VMEM((BW,), jnp.int32),
        pltpu.VMEM((BW,), jnp.float32),
        pltpu.VMEM((C, H), jnp.float32),
        pltpu.VMEM((C, H), jnp.float32),
        pltpu.SemaphoreType.DMA,
        pltpu.SemaphoreType.DMA,
        pltpu.SemaphoreType.DMA,
        pltpu.SemaphoreType.DMA,
    ],
    compiler_params=pltpu.CompilerParams(needs_layout_passes=False),
)(_scatter_body)


def _inv_body(cnt_ref, inv_ref):
    c = cnt_ref[0] + cnt_ref[1]
    inv_ref[...] = 1.0 / jnp.maximum(c, 1.0)


def _inv_counts(cnt2d):
    return pl.pallas_call(
        _inv_body,
        out_shape=jax.ShapeDtypeStruct((NRP,), jnp.float32),
    )(cnt2d)


def _ln_relu_res(acc0, acc1, root, bcv, gam, bet, hprev):
    s = acc0 + acc1 + root + bcv
    mu = jnp.mean(s, axis=-1, keepdims=True)
    var = jnp.mean((s - mu) ** 2, axis=-1, keepdims=True)
    s = (s - mu) * lax.rsqrt(var + 1e-5) * gam + bet
    return jnp.maximum(s, 0.0) + hprev


def _transforms(h, wrel_ref, wroot_ref, hr_ref, root_ref):
    hb = h.astype(jnp.bfloat16)
    for r in range(R):
        hr_ref[r] = jnp.dot(hb, wrel_ref[r].astype(jnp.bfloat16),
                            preferred_element_type=jnp.float32)
    root_ref[...] = jnp.dot(h, wroot_ref[...], preferred_element_type=jnp.float32)


def _dense_in_body(x_ref, wp_ref, bp_ref, wrel_ref, wroot_ref,
                   h_ref, hr_ref, root_ref):
    h = jnp.dot(x_ref[...], wp_ref[...],
                preferred_element_type=jnp.float32) + bp_ref[...]
    h_ref[...] = h
    _transforms(h, wrel_ref, wroot_ref, hr_ref, root_ref)


def _dense_mid_body(acc0_ref, acc1_ref, rootin_ref, hprev_ref, bcv_ref,
                    gam_ref, bet_ref, wrel_ref, wroot_ref,
                    h_ref, hr_ref, root_ref):
    h = _ln_relu_res(acc0_ref[...], acc1_ref[...], rootin_ref[...],
                     bcv_ref[...], gam_ref[...], bet_ref[...], hprev_ref[...])
    h_ref[...] = h
    _transforms(h, wrel_ref, wroot_ref, hr_ref, root_ref)


def _dense_out_body(acc0_ref, acc1_ref, rootin_ref, hprev_ref, bcv_ref,
                    gam_ref, bet_ref, batch_ref,
                    wm1_ref, bm1_ref, wm2_ref, bm2_ref,
                    wv1_ref, bv1_ref, wv2_ref, bv2_ref,
                    logits_ref, value_ref, g_acc):
    i = pl.program_id(0)

    @pl.when(i == 0)
    def _():
        g_acc[...] = jnp.zeros_like(g_acc)

    h = _ln_relu_res(acc0_ref[...], acc1_ref[...], rootin_ref[...],
                     bcv_ref[...], gam_ref[...], bet_ref[...], hprev_ref[...])
    b = batch_ref[...].reshape(NB)
    oh = (lax.broadcasted_iota(jnp.int32, (G, NB), 0) == b[None, :])
    g_acc[...] += jnp.dot(oh.astype(jnp.float32), h,
                          preferred_element_type=jnp.float32)

    @pl.when(i == pl.num_programs(0) - 1)
    def _():
        g = g_acc[...]
        hm = jnp.maximum(jnp.dot(g, wm1_ref[...],
                                 preferred_element_type=jnp.float32)
                         + bm1_ref[...], 0.0)
        logits_ref[...] = jnp.dot(hm, wm2_ref[...],
                                  preferred_element_type=jnp.float32) + bm2_ref[...]
        hv = jnp.maximum(jnp.dot(g, wv1_ref[...],
                                 preferred_element_type=jnp.float32)
                         + bv1_ref[...], 0.0)
        value_ref[...] = jnp.dot(hv, wv2_ref[...],
                                 preferred_element_type=jnp.float32) + bv2_ref[...]


_row_spec = pl.BlockSpec((NB, H), lambda i: (i, 0))
_vecH_spec = pl.BlockSpec((H,), lambda i: (0,))
_wrel_spec = pl.BlockSpec((R, H, H), lambda i: (0, 0, 0))
_wHH_spec = pl.BlockSpec((H, H), lambda i: (0, 0))
_hr_spec = pl.BlockSpec((R, NB, H), lambda i: (0, i, 0))

_h_hr_root_shapes = (jax.ShapeDtypeStruct((N, H), jnp.float32),
                     jax.ShapeDtypeStruct((R, N, H), jnp.float32),
                     jax.ShapeDtypeStruct((N, H), jnp.float32))


def _dense_in(x, Wp, bp, Wrel0, Wroot0):
    return pl.pallas_call(
        _dense_in_body,
        grid=(N // NB,),
        in_specs=[_row_spec, _wHH_spec, _vecH_spec, _wrel_spec, _wHH_spec],
        out_specs=[_row_spec, _hr_spec, _row_spec],
        out_shape=_h_hr_root_shapes,
    )(x, Wp, bp, Wrel0, Wroot0)


def _dense_mid(acc0, acc1, root, hprev, bcv, gam, bet, Wrel_l, Wroot_l):
    return pl.pallas_call(
        _dense_mid_body,
        grid=(N // NB,),
        in_specs=[_row_spec, _row_spec, _row_spec, _row_spec,
                  _vecH_spec, _vecH_spec, _vecH_spec, _wrel_spec, _wHH_spec],
        out_specs=[_row_spec, _hr_spec, _row_spec],
        out_shape=_h_hr_root_shapes,
    )(acc0, acc1, root, hprev, bcv, gam, bet, Wrel_l, Wroot_l)


def _dense_out(acc0, acc1, root, hprev, bcv, gam, bet, batch3d,
               Wm1, bm1, Wm2, bm2, Wv1, bv1, Wv2, bv2):
    wH = pl.BlockSpec((H, H), lambda i: (0, 0))
    wO = pl.BlockSpec((H, OUT), lambda i: (0, 0))
    vO = pl.BlockSpec((OUT,), lambda i: (0,))
    out_spec = pl.BlockSpec((G, OUT), lambda i: (0, 0))
    return pl.pallas_call(
        _dense_out_body,
        grid=(N // NB,),
        in_specs=[_row_spec, _row_spec, _row_spec, _row_spec,
                  _vecH_spec, _vecH_spec, _vecH_spec,
                  pl.BlockSpec((1, 1, NB), lambda i: (i, 0, 0)),
                  wH, _vecH_spec, wO, vO, wH, _vecH_spec, wO, vO],
        out_specs=[out_spec, out_spec],
        out_shape=(jax.ShapeDtypeStruct((G, OUT), jnp.float32),
                   jax.ShapeDtypeStruct((G, OUT), jnp.float32)),
        scratch_shapes=[pltpu.VMEM((G, H), jnp.float32)],
    )(acc0, acc1, root, hprev, bcv, gam, bet, batch3d,
      Wm1, bm1, Wm2, bm2, Wv1, bv1, Wv2, bv2)


def kernel(x, edge_index, edge_type, batch, Wp, bp, Wrel, Wroot, bconv,
           gamma, beta, Wm1, bm1, Wm2, bm2, Wv1, bv1, Wv2, bv2):
    src = edge_index[0].astype(jnp.int32)
    dst = edge_index[1].astype(jnp.int32)
    et = edge_type.astype(jnp.int32)
    batch3d = batch.astype(jnp.int32).reshape(N // NB, 1, NB)

    cnt = _count_edges(dst, et)
    inv = _inv_counts(cnt.reshape(2, NRP))
    w, gidx = _edge_weights(src, dst, et, inv)

    h, hr, root = _dense_in(x, Wp, bp, Wrel[0], Wroot[0])
    for l in range(L):
        accs = _edge_scatter(hr.reshape(R * N, H), gidx, dst, w)
        if l < L - 1:
            h, hr, root = _dense_mid(accs[0], accs[1], root, h, bconv[l],
                                     gamma[l], beta[l], Wrel[l + 1],
                                     Wroot[l + 1])
        else:
            logits, value = _dense_out(accs[0], accs[1], root, h, bconv[l],
                                       gamma[l], beta[l], batch3d,
                                       Wm1, bm1, Wm2, bm2, Wv1, bv1, Wv2, bv2)
    return (logits, value)
